# baseline (speedup 1.0000x reference)
import functools

import jax
import jax.numpy as jnp
from jax import lax
from jax.experimental import pallas as pl
from jax.experimental.pallas import tpu as pltpu
from jax.experimental.pallas import tpu_sc as plsc

B_, S_, V_ = 32, 16, 100000
R_ = B_ * S_
NW = 32
RPW = R_ // NW
CHUNK = 10000
NCHUNK = V_ // CHUNK
NVC = CHUNK // 16
CAP = 4096
CAP_EFF = CAP - 16
NSEL = 256
KSEL = 250
SAMP = 1024
EPS_C = 1e-10
TEMP_C = 0.1
IMIN = -2147483648
IMAX = 2147483647
NEG_INF = float("-inf")


def _iota16():
    return lax.iota(jnp.int32, 16)


def _take16(v, idx):
    dnums = lax.GatherDimensionNumbers(
        offset_dims=(), collapsed_slice_dims=(0,), start_index_map=(0,))
    return lax.gather(v, idx[:, None], dnums, (1,),
                      mode=lax.GatherScatterMode.PROMISE_IN_BOUNDS)


def _vmax16(v):
    idx = _iota16()
    for d in (8, 4, 2, 1):
        v = jnp.maximum(v, _take16(v, idx ^ d))
    return v[0]


def _cumsum16(v):
    idx = _iota16()
    for d in (1, 2, 4, 8):
        u = _take16(v, jnp.maximum(idx - d, 0))
        v = v + jnp.where(idx >= d, u, jnp.int32(0))
    return v


def _popcnt(m):
    return plsc.all_reduce_population_count(m)[0]


def _keys16(x):
    i = lax.bitcast_convert_type(x, jnp.int32)
    return jnp.where(i < 0, i ^ jnp.int32(0x7FFFFFFF), i)


def _key_to_float(k):
    kv = jnp.zeros((16,), jnp.int32) + k
    iv = jnp.where(kv < 0, kv ^ jnp.int32(0x7FFFFFFF), kv)
    return lax.bitcast_convert_type(iv, jnp.float32)[0]


def _float_to_key(f):
    fv = jnp.zeros((16,), jnp.float32) + f
    return _keys16(fv)[0]


def _sc_topk_call(logits_f, gx_f, cur_f):
    mesh = plsc.VectorSubcoreMesh(core_axis_name="c", subcore_axis_name="s")

    @functools.partial(
        pl.kernel,
        mesh=mesh,
        out_type=[
            jax.ShapeDtypeStruct((R_, NSEL), jnp.float32),
            jax.ShapeDtypeStruct((R_, NSEL), jnp.int32),
        ],
        compiler_params=pltpu.CompilerParams(needs_layout_passes=False),
        scratch_types=[
            pltpu.VMEM((CHUNK,), jnp.float32),
            pltpu.VMEM((CHUNK,), jnp.float32),
            pltpu.VMEM((SAMP,), jnp.float32),
            pltpu.VMEM((CAP,), jnp.float32),
            pltpu.VMEM((CAP,), jnp.int32),
            pltpu.VMEM((CAP,), jnp.int32),
            pltpu.VMEM((16 * 256,), jnp.int32),
            pltpu.VMEM((272,), jnp.int32),
            pltpu.VMEM((NSEL,), jnp.int32),
            pltpu.VMEM((NSEL,), jnp.int32),
            pltpu.VMEM((NSEL,), jnp.float32),
            pltpu.VMEM((NSEL,), jnp.int32),
            pltpu.VMEM((NSEL,), jnp.int32),
            pltpu.VMEM((NSEL,), jnp.float32),
            pltpu.VMEM((2, 128), jnp.int32),
            pltpu.VMEM((NSEL,), jnp.float32),
            pltpu.VMEM((RPW + 16,), jnp.int32),
            pltpu.SemaphoreType.DMA,
            pltpu.SemaphoreType.DMA,
            pltpu.SemaphoreType.DMA,
        ],
    )
    def k(logits_hbm, gx_hbm, cur_hbm, prop_hbm, id_hbm,
          cbuf0, cbuf1, samp, cval, cidx, ckey, hist, sfx,
          ask, aid, aprop, bsk, bid, bprop, gid, gxb, curb,
          sem0, sem1, semg):
        wid = lax.axis_index("s") * 2 + lax.axis_index("c")
        pltpu.sync_copy(cur_hbm.at[pl.ds(wid * RPW, RPW)],
                        curb.at[pl.ds(0, RPW)])
        lane = _iota16()

        def row_body(rl, _carry):
            row = wid * RPW + rl
            row_base = row * V_

            pltpu.sync_copy(logits_hbm.at[pl.ds(row_base, SAMP)], samp)

            def ext_one(_t, _thr):
                def mx(j, acc):
                    return jnp.maximum(acc, samp[pl.ds(j * 16, 16)])
                acc = lax.fori_loop(0, SAMP // 16, mx,
                                    jnp.full((16,), NEG_INF, jnp.float32))
                m = _vmax16(acc)

                def rm(j, c):
                    v = samp[pl.ds(j * 16, 16)]
                    samp[pl.ds(j * 16, 16)] = jnp.where(
                        v == m, jnp.float32(NEG_INF), v)
                    return c
                lax.fori_loop(0, SAMP // 16, rm, 0)
                return m
            thr0 = lax.fori_loop(0, 10, ext_one, jnp.float32(0.0))

            def start_chunk(c, buf, sem):
                pltpu.make_async_copy(
                    logits_hbm.at[pl.ds(row_base + c * CHUNK, CHUNK)],
                    buf, sem).start()

            def wait_chunk(buf, sem):
                pltpu.make_async_copy(
                    logits_hbm.at[pl.ds(row_base, CHUNK)],
                    buf, sem).wait()

            def scan_row(thr_f):
                start_chunk(0, cbuf0, sem0)

                def process(buf, base, cnt):
                    def vb(j, c):
                        x = buf[pl.ds(j * 16, 16)]
                        m = x >= thr_f
                        cnt_in = _popcnt(m)

                        @pl.when(cnt_in > 0)
                        def _():
                            skey = jnp.where(m, lane, lane + 16)
                            pos = jnp.minimum(c, CAP - 16)
                            cval[pl.ds(pos, 16)] = (
                                plsc.sort_key_val(skey, x)[1])
                            idxv = (base + j * 16) + lane
                            cidx[pl.ds(pos, 16)] = (
                                plsc.sort_key_val(skey, idxv)[1])
                        return c + cnt_in
                    return lax.fori_loop(0, NVC, vb, cnt)

                def super_body(t, cnt):
                    c0 = 2 * t
                    wait_chunk(cbuf0, sem0)
                    start_chunk(c0 + 1, cbuf1, sem1)
                    cnt = process(cbuf0, c0 * CHUNK, cnt)
                    wait_chunk(cbuf1, sem1)

                    @pl.when(t < NCHUNK // 2 - 1)
                    def _():
                        start_chunk(c0 + 2, cbuf0, sem0)
                    cnt = process(cbuf1, (c0 + 1) * CHUNK, cnt)
                    return cnt
                return lax.fori_loop(0, NCHUNK // 2, super_body, 0)

            cnt0 = scan_row(thr0)

            def bad(c):
                _thr, klo, khi, cnt = c
                out_of_range = (cnt < KSEL) | (cnt > CAP_EFF)
                return out_of_range & (khi > klo + 1)

            def rescan(c):
                thr, klo, khi, cnt = c
                kt = _float_to_key(thr)
                klo2 = jnp.where(cnt > CAP_EFF, kt, klo)
                khi2 = jnp.where(cnt < KSEL, kt, khi)
                kmid = (klo2 >> 1) + (khi2 >> 1) + (klo2 & khi2 & 1)
                thr2 = _key_to_float(kmid)
                cnt2 = scan_row(thr2)
                return thr2, klo2, khi2, cnt2

            _thr, _klo, _khi, n = lax.while_loop(
                bad, rescan,
                (thr0, jnp.int32(IMIN), jnp.int32(IMAX), cnt0))
            nv = (n + 15) >> 4

            def kb(j, c):
                x = cval[pl.ds(j * 16, 16)]
                ks = _keys16(x)
                valid = (j * 16 + lane) < n
                ckey[pl.ds(j * 16, 16)] = jnp.where(valid, ks, jnp.int32(IMIN))
                return c
            lax.fori_loop(0, nv, kb, 0)

            lanebase = lane * 256
            prefix = jnp.int32(0)
            r_cur = jnp.int32(KSEL)
            cnt_gt = jnp.int32(0)
            for lvl in range(4):
                shift = 24 - 8 * lvl

                def zb(j, c):
                    hist[pl.ds(j * 16, 16)] = jnp.zeros((16,), jnp.int32)
                    return c
                lax.fori_loop(0, 256, zb, 0)

                pfx = prefix

                def hb(j, c, _lvl=lvl, _shift=shift, _pfx=pfx):
                    ks = ckey[pl.ds(j * 16, 16)]
                    u = ks ^ jnp.int32(IMIN)
                    b = lax.shift_right_logical(u, _shift) & 255
                    m = (j * 16 + lane) < n
                    if _lvl > 0:
                        m = m & (lax.shift_right_logical(u, _shift + 8) == _pfx)
                    plsc.addupdate_scatter(
                        hist, [lanebase + b], jnp.ones((16,), jnp.int32),
                        mask=m)
                    return c
                lax.fori_loop(0, nv, hb, 0)

                sfx[pl.ds(256, 16)] = jnp.zeros((16,), jnp.int32)

                def tb(t, carry):
                    cchunk = 15 - t

                    def lb(ln, a):
                        return a + hist[pl.ds(ln * 256 + cchunk * 16, 16)]
                    acc = lax.fori_loop(0, 16, lb, jnp.zeros((16,), jnp.int32))
                    cs = _cumsum16(lax.rev(acc, (0,))) + carry
                    sfx[pl.ds(cchunk * 16, 16)] = lax.rev(cs, (0,))
                    return cs[15]
                lax.fori_loop(0, 16, tb, jnp.int32(0))

                def fb(ci, bstar):
                    v = sfx[pl.ds(ci * 16, 16)]
                    idxv = ci * 16 + lane
                    sel = jnp.where(v >= r_cur, idxv, jnp.int32(-1))
                    return jnp.maximum(bstar, _vmax16(sel))
                bstar = lax.fori_loop(0, 16, fb, jnp.int32(-1))
                bstar = jnp.maximum(bstar, 0)
                gt_above = sfx[pl.ds(bstar + 1, 16)][0]
                cnt_gt = cnt_gt + gt_above
                r_cur = r_cur - gt_above
                prefix = (prefix << 8) | bstar

            t_ks = prefix ^ jnp.int32(IMIN)
            eq_need = jnp.int32(KSEL) - cnt_gt

            def sb(j, carry):
                cursor, eq_carry = carry
                ks = ckey[pl.ds(j * 16, 16)]
                valid = (j * 16 + lane) < n
                gt = (ks > t_ks) & valid
                eq = (ks == t_ks) & valid
                ecs = _cumsum16(eq.astype(jnp.int32))
                sel = gt | (eq & ((ecs + eq_carry) <= eq_need))
                seli = sel.astype(jnp.int32)
                dst = cursor + _cumsum16(seli) - seli
                tpos = (dst & 15) * 16 + lax.shift_right_logical(dst, 4)
                sk = ks ^ jnp.int32(0x7FFFFFFF)
                plsc.store_scatter(ask, [tpos], sk, mask=sel)
                plsc.store_scatter(aid, [tpos], cidx[pl.ds(j * 16, 16)],
                                   mask=sel)
                return (cursor + _popcnt(sel), eq_carry + _popcnt(eq))
            lax.fori_loop(0, nv, sb, (jnp.int32(0), jnp.int32(0)))

            padidx = jnp.minimum(175 + lane * 16, 255)
            padm = lane < 6
            plsc.store_scatter(ask, [padidx],
                               jnp.full((16,), -1, jnp.int32), mask=padm)
            plsc.store_scatter(aid, [padidx],
                               jnp.zeros((16,), jnp.int32), mask=padm)

            for half in range(2):
                def gbj(j, c, _h=half):
                    idv = aid[pl.ds(_h * 128 + j * 16, 16)]
                    gid[_h, pl.ds(j * 16, 16)] = idv + row_base
                    return c
                lax.fori_loop(0, 8, gbj, 0)
            cp0 = pltpu.make_async_copy(
                gx_hbm.at[gid.at[0]], gxb.at[pl.ds(0, 128)], semg)
            cp0.start()
            cp1 = pltpu.make_async_copy(
                gx_hbm.at[gid.at[1]], gxb.at[pl.ds(128, 128)], semg)
            cp1.start()
            cp0.wait()
            cp1.wait()

            curtok = curb[pl.ds(rl, 16)][0]

            def pb(j, c):
                idv = aid[pl.ds(j * 16, 16)]
                g = gxb[pl.ds(j * 16, 16)]
                sc = jnp.where(idv == curtok, jnp.float32(EPS_C),
                               jnp.float32(1.0))
                aprop[pl.ds(j * 16, 16)] = (-(g * sc)) / jnp.float32(TEMP_C)
                return c
            lax.fori_loop(0, NSEL // 16, pb, 0)

            bufs = [(ask, aid, aprop), (bsk, bid, bprop)]
            for p in range(8):
                src3 = bufs[p % 2]
                dst3 = bufs[(p + 1) % 2]
                sh = 4 * p
                last = p == 7

                def zb2(t, c):
                    hist[pl.ds(t * 16, 16)] = jnp.zeros((16,), jnp.int32)
                    return c
                lax.fori_loop(0, 16, zb2, 0)

                def hb2(j, c, _sh=sh, _src=src3[0]):
                    sk = _src[pl.ds(j * 16, 16)]
                    d = lax.shift_right_logical(sk, _sh) & 15
                    plsc.addupdate_scatter(hist, [d * 16 + lane],
                                           jnp.ones((16,), jnp.int32))
                    return c
                lax.fori_loop(0, 16, hb2, 0)

                def pb2(t, carry):
                    v = hist[pl.ds(t * 16, 16)]
                    inc = _cumsum16(v)
                    hist[pl.ds(t * 16, 16)] = inc - v + carry
                    return carry + inc[15]
                lax.fori_loop(0, 16, pb2, jnp.int32(0))

                def mb(j, c, _sh=sh, _s=src3, _d=dst3, _last=last):
                    sk = _s[0][pl.ds(j * 16, 16)]
                    dg = lax.shift_right_logical(sk, _sh) & 15
                    addr = dg * 16 + lane
                    pp = plsc.load_gather(hist, [addr])
                    plsc.store_scatter(hist, [addr], pp + 1)
                    if _last:
                        tpos = pp
                    else:
                        tpos = (pp & 15) * 16 + lax.shift_right_logical(pp, 4)
                    plsc.store_scatter(_d[0], [tpos], sk)
                    plsc.store_scatter(_d[1], [tpos],
                                       _s[1][pl.ds(j * 16, 16)])
                    plsc.store_scatter(_d[2], [tpos],
                                       _s[2][pl.ds(j * 16, 16)])
                    return c
                lax.fori_loop(0, 16, mb, 0)

            pltpu.sync_copy(aprop.at[pl.ds(0, NSEL)], prop_hbm.at[row])
            pltpu.sync_copy(aid.at[pl.ds(0, NSEL)], id_hbm.at[row])
            return _carry

        lax.fori_loop(0, RPW, row_body, 0)

    return k(logits_f, gx_f, cur_f)


def kernel(gx, logits, cur_token_ids):
    logits_f = logits.reshape(-1)
    gx_f = gx.reshape(-1)
    cur_f = cur_token_ids.reshape(-1).astype(jnp.int32)
    prop, ids = _sc_topk_call(logits_f, gx_f, cur_f)
    proposal = prop[:, :KSEL].reshape(B_, S_, KSEL)
    topk_ids = ids[:, :KSEL].reshape(B_, S_, KSEL)
    return (proposal, topk_ids)


# speedup vs baseline: 2.9214x; 2.9214x over previous
import functools

import jax
import jax.numpy as jnp
from jax import lax
from jax.experimental import pallas as pl
from jax.experimental.pallas import tpu as pltpu
from jax.experimental.pallas import tpu_sc as plsc

B_, S_, V_ = 32, 16, 100000
R_ = B_ * S_
NW = 32
RPW = R_ // NW
CHUNK = 10000
NCHUNK = V_ // CHUNK
NVC = CHUNK // 16
CAP = 4096
CAP_EFF = CAP - 16
NSEL = 256
KSEL = 250
SAMP = 1024
EPS_C = 1e-10
TEMP_C = 0.1
IMIN = -2147483648
IMAX = 2147483647
NEG_INF = float("-inf")


def _iota16():
    return lax.iota(jnp.int32, 16)


def _take16(v, idx):
    dnums = lax.GatherDimensionNumbers(
        offset_dims=(), collapsed_slice_dims=(0,), start_index_map=(0,))
    return lax.gather(v, idx[:, None], dnums, (1,),
                      mode=lax.GatherScatterMode.PROMISE_IN_BOUNDS)


def _vmax16(v):
    idx = _iota16()
    for d in (8, 4, 2, 1):
        v = jnp.maximum(v, _take16(v, idx ^ d))
    return v[0]


def _cumsum16(v):
    idx = _iota16()
    for d in (1, 2, 4, 8):
        u = _take16(v, jnp.maximum(idx - d, 0))
        v = v + jnp.where(idx >= d, u, jnp.int32(0))
    return v


def _popcnt(m):
    return plsc.all_reduce_population_count(m)[0]


def _keys16(x):
    i = lax.bitcast_convert_type(x, jnp.int32)
    return jnp.where(i < 0, i ^ jnp.int32(0x7FFFFFFF), i)


def _key_to_float(k):
    kv = jnp.zeros((16,), jnp.int32) + k
    iv = jnp.where(kv < 0, kv ^ jnp.int32(0x7FFFFFFF), kv)
    return lax.bitcast_convert_type(iv, jnp.float32)[0]


def _float_to_key(f):
    fv = jnp.zeros((16,), jnp.float32) + f
    return _keys16(fv)[0]


def _sc_topk_call(logits_f, gx_f, cur_f):
    mesh = plsc.VectorSubcoreMesh(core_axis_name="c", subcore_axis_name="s")

    @functools.partial(
        pl.kernel,
        mesh=mesh,
        out_type=[
            jax.ShapeDtypeStruct((R_, NSEL), jnp.float32),
            jax.ShapeDtypeStruct((R_, NSEL), jnp.int32),
        ],
        compiler_params=pltpu.CompilerParams(needs_layout_passes=False),
        scratch_types=[
            pltpu.VMEM((CHUNK,), jnp.float32),
            pltpu.VMEM((CHUNK,), jnp.float32),
            pltpu.VMEM((SAMP,), jnp.float32),
            pltpu.VMEM((CAP,), jnp.float32),
            pltpu.VMEM((CAP,), jnp.int32),
            pltpu.VMEM((CAP,), jnp.int32),
            pltpu.VMEM((16 * 256,), jnp.int32),
            pltpu.VMEM((272,), jnp.int32),
            pltpu.VMEM((NSEL,), jnp.int32),
            pltpu.VMEM((NSEL,), jnp.int32),
            pltpu.VMEM((NSEL,), jnp.float32),
            pltpu.VMEM((NSEL,), jnp.int32),
            pltpu.VMEM((NSEL,), jnp.int32),
            pltpu.VMEM((NSEL,), jnp.float32),
            pltpu.VMEM((2, 128), jnp.int32),
            pltpu.VMEM((NSEL,), jnp.float32),
            pltpu.VMEM((RPW + 16,), jnp.int32),
            pltpu.SemaphoreType.DMA,
            pltpu.SemaphoreType.DMA,
            pltpu.SemaphoreType.DMA,
        ],
    )
    def k(logits_hbm, gx_hbm, cur_hbm, prop_hbm, id_hbm,
          cbuf0, cbuf1, samp, cval, cidx, ckey, hist, sfx,
          ask, aid, aprop, bsk, bid, bprop, gid, gxb, curb,
          sem0, sem1, semg):
        wid = lax.axis_index("s") * 2 + lax.axis_index("c")
        pltpu.sync_copy(cur_hbm.at[pl.ds(wid * RPW, RPW)],
                        curb.at[pl.ds(0, RPW)])
        lane = _iota16()

        def row_body(rl, _carry):
            row = wid * RPW + rl
            row_base = row * V_

            pltpu.sync_copy(logits_hbm.at[pl.ds(row_base, SAMP)], samp)

            def ext_one(_t, _thr):
                def mx(j, acc):
                    return jnp.maximum(acc, samp[pl.ds(j * 16, 16)])
                acc = lax.fori_loop(0, SAMP // 16, mx,
                                    jnp.full((16,), NEG_INF, jnp.float32))
                m = _vmax16(acc)

                def rm(j, c):
                    v = samp[pl.ds(j * 16, 16)]
                    samp[pl.ds(j * 16, 16)] = jnp.where(
                        v == m, jnp.float32(NEG_INF), v)
                    return c
                lax.fori_loop(0, SAMP // 16, rm, 0)
                return m
            thr0 = lax.fori_loop(0, 10, ext_one, jnp.float32(0.0))

            def start_chunk(c, buf, sem):
                pltpu.make_async_copy(
                    logits_hbm.at[pl.ds(row_base + c * CHUNK, CHUNK)],
                    buf, sem).start()

            def wait_chunk(buf, sem):
                pltpu.make_async_copy(
                    logits_hbm.at[pl.ds(row_base, CHUNK)],
                    buf, sem).wait()

            def scan_row(thr_f):
                start_chunk(0, cbuf0, sem0)

                def process(buf, base, cnt):
                    def vb(j, c):
                        x = buf[pl.ds(j * 16, 16)]
                        m = x >= thr_f
                        skey = jnp.where(m, lane, lane + 16)
                        sv = plsc.sort_key_val(skey, x)[1]
                        idxv = (base + j * 16) + lane
                        si = plsc.sort_key_val(skey, idxv)[1]
                        cnt_in = _popcnt(m)
                        pos = jnp.minimum(c, CAP - 16)
                        dmask = lane < cnt_in
                        plsc.store_scatter(cval, [pos + lane], sv, mask=dmask)
                        plsc.store_scatter(cidx, [pos + lane], si, mask=dmask)
                        return c + cnt_in
                    return plsc.parallel_loop(0, NVC, unroll=4, carry=cnt)(vb)

                def super_body(t, cnt):
                    c0 = 2 * t
                    wait_chunk(cbuf0, sem0)
                    start_chunk(c0 + 1, cbuf1, sem1)
                    cnt = process(cbuf0, c0 * CHUNK, cnt)
                    wait_chunk(cbuf1, sem1)

                    @pl.when(t < NCHUNK // 2 - 1)
                    def _():
                        start_chunk(c0 + 2, cbuf0, sem0)
                    cnt = process(cbuf1, (c0 + 1) * CHUNK, cnt)
                    return cnt
                return lax.fori_loop(0, NCHUNK // 2, super_body, 0)

            cnt0 = scan_row(thr0)

            def bad(c):
                _thr, klo, khi, cnt = c
                out_of_range = (cnt < KSEL) | (cnt > CAP_EFF)
                return out_of_range & (khi > klo + 1)

            def rescan(c):
                thr, klo, khi, cnt = c
                kt = _float_to_key(thr)
                klo2 = jnp.where(cnt > CAP_EFF, kt, klo)
                khi2 = jnp.where(cnt < KSEL, kt, khi)
                kmid = (klo2 >> 1) + (khi2 >> 1) + (klo2 & khi2 & 1)
                thr2 = _key_to_float(kmid)
                cnt2 = scan_row(thr2)
                return thr2, klo2, khi2, cnt2

            _thr, _klo, _khi, n = lax.while_loop(
                bad, rescan,
                (thr0, jnp.int32(IMIN), jnp.int32(IMAX), cnt0))
            nv = (n + 15) >> 4

            def kb(j, c):
                x = cval[pl.ds(j * 16, 16)]
                ks = _keys16(x)
                valid = (j * 16 + lane) < n
                ckey[pl.ds(j * 16, 16)] = jnp.where(valid, ks, jnp.int32(IMIN))
                return c
            lax.fori_loop(0, nv, kb, 0)

            lanebase = lane * 256
            prefix = jnp.int32(0)
            r_cur = jnp.int32(KSEL)
            cnt_gt = jnp.int32(0)
            for lvl in range(4):
                shift = 24 - 8 * lvl

                def zb(j, c):
                    hist[pl.ds(j * 16, 16)] = jnp.zeros((16,), jnp.int32)
                    return c
                lax.fori_loop(0, 256, zb, 0)

                pfx = prefix

                def hb(j, c, _lvl=lvl, _shift=shift, _pfx=pfx):
                    ks = ckey[pl.ds(j * 16, 16)]
                    u = ks ^ jnp.int32(IMIN)
                    b = lax.shift_right_logical(u, _shift) & 255
                    m = (j * 16 + lane) < n
                    if _lvl > 0:
                        m = m & (lax.shift_right_logical(u, _shift + 8) == _pfx)
                    plsc.addupdate_scatter(
                        hist, [lanebase + b], jnp.ones((16,), jnp.int32),
                        mask=m)
                    return c
                lax.fori_loop(0, nv, hb, 0)

                sfx[pl.ds(256, 16)] = jnp.zeros((16,), jnp.int32)

                def tb(t, carry):
                    cchunk = 15 - t

                    def lb(ln, a):
                        return a + hist[pl.ds(ln * 256 + cchunk * 16, 16)]
                    acc = lax.fori_loop(0, 16, lb, jnp.zeros((16,), jnp.int32))
                    cs = _cumsum16(lax.rev(acc, (0,))) + carry
                    sfx[pl.ds(cchunk * 16, 16)] = lax.rev(cs, (0,))
                    return cs[15]
                lax.fori_loop(0, 16, tb, jnp.int32(0))

                def fb(ci, bstar):
                    v = sfx[pl.ds(ci * 16, 16)]
                    idxv = ci * 16 + lane
                    sel = jnp.where(v >= r_cur, idxv, jnp.int32(-1))
                    return jnp.maximum(bstar, _vmax16(sel))
                bstar = lax.fori_loop(0, 16, fb, jnp.int32(-1))
                bstar = jnp.maximum(bstar, 0)
                gt_above = sfx[pl.ds(bstar + 1, 16)][0]
                cnt_gt = cnt_gt + gt_above
                r_cur = r_cur - gt_above
                prefix = (prefix << 8) | bstar

            t_ks = prefix ^ jnp.int32(IMIN)
            eq_need = jnp.int32(KSEL) - cnt_gt

            def sb(j, carry):
                cursor, eq_carry = carry
                ks = ckey[pl.ds(j * 16, 16)]
                valid = (j * 16 + lane) < n
                gt = (ks > t_ks) & valid
                eq = (ks == t_ks) & valid
                ecs = _cumsum16(eq.astype(jnp.int32))
                sel = gt | (eq & ((ecs + eq_carry) <= eq_need))
                seli = sel.astype(jnp.int32)
                dst = cursor + _cumsum16(seli) - seli
                tpos = (dst & 15) * 16 + lax.shift_right_logical(dst, 4)
                sk = ks ^ jnp.int32(0x7FFFFFFF)
                plsc.store_scatter(ask, [tpos], sk, mask=sel)
                plsc.store_scatter(aid, [tpos], cidx[pl.ds(j * 16, 16)],
                                   mask=sel)
                return (cursor + _popcnt(sel), eq_carry + _popcnt(eq))
            lax.fori_loop(0, nv, sb, (jnp.int32(0), jnp.int32(0)))

            padidx = jnp.minimum(175 + lane * 16, 255)
            padm = lane < 6
            plsc.store_scatter(ask, [padidx],
                               jnp.full((16,), -1, jnp.int32), mask=padm)
            plsc.store_scatter(aid, [padidx],
                               jnp.zeros((16,), jnp.int32), mask=padm)

            for half in range(2):
                def gbj(j, c, _h=half):
                    idv = aid[pl.ds(_h * 128 + j * 16, 16)]
                    gid[_h, pl.ds(j * 16, 16)] = idv + row_base
                    return c
                lax.fori_loop(0, 8, gbj, 0)
            cp0 = pltpu.make_async_copy(
                gx_hbm.at[gid.at[0]], gxb.at[pl.ds(0, 128)], semg)
            cp0.start()
            cp1 = pltpu.make_async_copy(
                gx_hbm.at[gid.at[1]], gxb.at[pl.ds(128, 128)], semg)
            cp1.start()
            cp0.wait()
            cp1.wait()

            curtok = curb[pl.ds(rl, 16)][0]

            def pb(j, c):
                idv = aid[pl.ds(j * 16, 16)]
                g = gxb[pl.ds(j * 16, 16)]
                sc = jnp.where(idv == curtok, jnp.float32(EPS_C),
                               jnp.float32(1.0))
                aprop[pl.ds(j * 16, 16)] = (-(g * sc)) / jnp.float32(TEMP_C)
                return c
            lax.fori_loop(0, NSEL // 16, pb, 0)

            bufs = [(ask, aid, aprop), (bsk, bid, bprop)]
            for p in range(8):
                src3 = bufs[p % 2]
                dst3 = bufs[(p + 1) % 2]
                sh = 4 * p
                last = p == 7

                def zb2(t, c):
                    hist[pl.ds(t * 16, 16)] = jnp.zeros((16,), jnp.int32)
                    return c
                lax.fori_loop(0, 16, zb2, 0)

                def hb2(j, c, _sh=sh, _src=src3[0]):
                    sk = _src[pl.ds(j * 16, 16)]
                    d = lax.shift_right_logical(sk, _sh) & 15
                    plsc.addupdate_scatter(hist, [d * 16 + lane],
                                           jnp.ones((16,), jnp.int32))
                    return c
                lax.fori_loop(0, 16, hb2, 0)

                def pb2(t, carry):
                    v = hist[pl.ds(t * 16, 16)]
                    inc = _cumsum16(v)
                    hist[pl.ds(t * 16, 16)] = inc - v + carry
                    return carry + inc[15]
                lax.fori_loop(0, 16, pb2, jnp.int32(0))

                def mb(j, c, _sh=sh, _s=src3, _d=dst3, _last=last):
                    sk = _s[0][pl.ds(j * 16, 16)]
                    dg = lax.shift_right_logical(sk, _sh) & 15
                    addr = dg * 16 + lane
                    pp = plsc.load_gather(hist, [addr])
                    plsc.store_scatter(hist, [addr], pp + 1)
                    if _last:
                        tpos = pp
                    else:
                        tpos = (pp & 15) * 16 + lax.shift_right_logical(pp, 4)
                    plsc.store_scatter(_d[0], [tpos], sk)
                    plsc.store_scatter(_d[1], [tpos],
                                       _s[1][pl.ds(j * 16, 16)])
                    plsc.store_scatter(_d[2], [tpos],
                                       _s[2][pl.ds(j * 16, 16)])
                    return c
                lax.fori_loop(0, 16, mb, 0)

            pltpu.sync_copy(aprop.at[pl.ds(0, NSEL)], prop_hbm.at[row])
            pltpu.sync_copy(aid.at[pl.ds(0, NSEL)], id_hbm.at[row])
            return _carry

        lax.fori_loop(0, RPW, row_body, 0)

    return k(logits_f, gx_f, cur_f)


def kernel(gx, logits, cur_token_ids):
    logits_f = logits.reshape(-1)
    gx_f = gx.reshape(-1)
    cur_f = cur_token_ids.reshape(-1).astype(jnp.int32)
    prop, ids = _sc_topk_call(logits_f, gx_f, cur_f)
    proposal = prop[:, :KSEL].reshape(B_, S_, KSEL)
    topk_ids = ids[:, :KSEL].reshape(B_, S_, KSEL)
    return (proposal, topk_ids)


# speedup vs baseline: 2.9537x; 1.0111x over previous
import functools

import jax
import jax.numpy as jnp
from jax import lax
from jax.experimental import pallas as pl
from jax.experimental.pallas import tpu as pltpu
from jax.experimental.pallas import tpu_sc as plsc

B_, S_, V_ = 32, 16, 100000
R_ = B_ * S_
NW = 32
RPW = R_ // NW
CHUNK = 10000
NCHUNK = V_ // CHUNK
NVC = CHUNK // 16
CAP = 4096
CAP_EFF = CAP - 16
NSEL = 256
KSEL = 250
SAMP = 1024
EPS_C = 1e-10
TEMP_C = 0.1
IMIN = -2147483648
IMAX = 2147483647
NEG_INF = float("-inf")


def _iota16():
    return lax.iota(jnp.int32, 16)


def _take16(v, idx):
    dnums = lax.GatherDimensionNumbers(
        offset_dims=(), collapsed_slice_dims=(0,), start_index_map=(0,))
    return lax.gather(v, idx[:, None], dnums, (1,),
                      mode=lax.GatherScatterMode.PROMISE_IN_BOUNDS)


def _vmax16(v):
    idx = _iota16()
    for d in (8, 4, 2, 1):
        v = jnp.maximum(v, _take16(v, idx ^ d))
    return v[0]


def _cumsum16(v):
    idx = _iota16()
    for d in (1, 2, 4, 8):
        u = _take16(v, jnp.maximum(idx - d, 0))
        v = v + jnp.where(idx >= d, u, jnp.int32(0))
    return v


def _popcnt(m):
    return plsc.all_reduce_population_count(m)[0]


def _keys16(x):
    i = lax.bitcast_convert_type(x, jnp.int32)
    return jnp.where(i < 0, i ^ jnp.int32(0x7FFFFFFF), i)


def _key_to_float(k):
    kv = jnp.zeros((16,), jnp.int32) + k
    iv = jnp.where(kv < 0, kv ^ jnp.int32(0x7FFFFFFF), kv)
    return lax.bitcast_convert_type(iv, jnp.float32)[0]


def _float_to_key(f):
    fv = jnp.zeros((16,), jnp.float32) + f
    return _keys16(fv)[0]


def _sc_topk_call(logits_f, gx_f, cur_f):
    mesh = plsc.VectorSubcoreMesh(core_axis_name="c", subcore_axis_name="s")

    @functools.partial(
        pl.kernel,
        mesh=mesh,
        out_type=[
            jax.ShapeDtypeStruct((R_, NSEL), jnp.float32),
            jax.ShapeDtypeStruct((R_, NSEL), jnp.int32),
        ],
        compiler_params=pltpu.CompilerParams(needs_layout_passes=False),
        scratch_types=[
            pltpu.VMEM((CHUNK,), jnp.float32),
            pltpu.VMEM((CHUNK,), jnp.float32),
            pltpu.VMEM((SAMP,), jnp.float32),
            pltpu.VMEM((CAP,), jnp.float32),
            pltpu.VMEM((CAP,), jnp.int32),
            pltpu.VMEM((CAP,), jnp.int32),
            pltpu.VMEM((16 * 256,), jnp.int32),
            pltpu.VMEM((272,), jnp.int32),
            pltpu.VMEM((NSEL,), jnp.int32),
            pltpu.VMEM((NSEL,), jnp.int32),
            pltpu.VMEM((NSEL,), jnp.float32),
            pltpu.VMEM((NSEL,), jnp.int32),
            pltpu.VMEM((NSEL,), jnp.int32),
            pltpu.VMEM((NSEL,), jnp.float32),
            pltpu.VMEM((2, 128), jnp.int32),
            pltpu.VMEM((NSEL,), jnp.float32),
            pltpu.VMEM((RPW + 16,), jnp.int32),
            pltpu.SemaphoreType.DMA,
            pltpu.SemaphoreType.DMA,
            pltpu.SemaphoreType.DMA,
        ],
    )
    def k(logits_hbm, gx_hbm, cur_hbm, prop_hbm, id_hbm,
          cbuf0, cbuf1, samp, cval, cidx, ckey, hist, sfx,
          ask, aid, aprop, bsk, bid, bprop, gid, gxb, curb,
          sem0, sem1, semg):
        wid = lax.axis_index("s") * 2 + lax.axis_index("c")
        pltpu.sync_copy(cur_hbm.at[pl.ds(wid * RPW, RPW)],
                        curb.at[pl.ds(0, RPW)])
        lane = _iota16()

        def row_body(rl, _carry):
            row = wid * RPW + rl
            row_base = row * V_

            pltpu.sync_copy(logits_hbm.at[pl.ds(row_base, SAMP)], samp)

            def ext_one(_t, _thr):
                def mx(j, acc):
                    return jnp.maximum(acc, samp[pl.ds(j * 16, 16)])
                acc = lax.fori_loop(0, SAMP // 16, mx,
                                    jnp.full((16,), NEG_INF, jnp.float32))
                m = _vmax16(acc)

                def rm(j, c):
                    v = samp[pl.ds(j * 16, 16)]
                    samp[pl.ds(j * 16, 16)] = jnp.where(
                        v == m, jnp.float32(NEG_INF), v)
                    return c
                lax.fori_loop(0, SAMP // 16, rm, 0)
                return m
            thr0 = lax.fori_loop(0, 10, ext_one, jnp.float32(0.0))

            def start_chunk(c, buf, sem):
                pltpu.make_async_copy(
                    logits_hbm.at[pl.ds(row_base + c * CHUNK, CHUNK)],
                    buf, sem).start()

            def wait_chunk(buf, sem):
                pltpu.make_async_copy(
                    logits_hbm.at[pl.ds(row_base, CHUNK)],
                    buf, sem).wait()

            def scan_row(thr_f):
                start_chunk(0, cbuf0, sem0)

                def process(buf, base, cnt):
                    def vb(j, c):
                        x = buf[pl.ds(j * 16, 16)]
                        m = x >= thr_f
                        skey = jnp.where(m, lane, lane + 16)
                        sv = plsc.sort_key_val(skey, x)[1]
                        idxv = (base + j * 16) + lane
                        si = plsc.sort_key_val(skey, idxv)[1]
                        cnt_in = _popcnt(m)
                        pos = jnp.minimum(c, CAP - 16)
                        dmask = lane < cnt_in
                        plsc.store_scatter(cval, [pos + lane], sv, mask=dmask)
                        plsc.store_scatter(cidx, [pos + lane], si, mask=dmask)
                        return c + cnt_in
                    return plsc.parallel_loop(0, NVC, unroll=8, carry=cnt)(vb)

                def super_body(t, cnt):
                    c0 = 2 * t
                    wait_chunk(cbuf0, sem0)
                    start_chunk(c0 + 1, cbuf1, sem1)
                    cnt = process(cbuf0, c0 * CHUNK, cnt)
                    wait_chunk(cbuf1, sem1)

                    @pl.when(t < NCHUNK // 2 - 1)
                    def _():
                        start_chunk(c0 + 2, cbuf0, sem0)
                    cnt = process(cbuf1, (c0 + 1) * CHUNK, cnt)
                    return cnt
                return lax.fori_loop(0, NCHUNK // 2, super_body, 0)

            cnt0 = scan_row(thr0)

            def bad(c):
                _thr, klo, khi, cnt = c
                out_of_range = (cnt < KSEL) | (cnt > CAP_EFF)
                return out_of_range & (khi > klo + 1)

            def rescan(c):
                thr, klo, khi, cnt = c
                kt = _float_to_key(thr)
                klo2 = jnp.where(cnt > CAP_EFF, kt, klo)
                khi2 = jnp.where(cnt < KSEL, kt, khi)
                kmid = (klo2 >> 1) + (khi2 >> 1) + (klo2 & khi2 & 1)
                thr2 = _key_to_float(kmid)
                cnt2 = scan_row(thr2)
                return thr2, klo2, khi2, cnt2

            _thr, _klo, _khi, n = lax.while_loop(
                bad, rescan,
                (thr0, jnp.int32(IMIN), jnp.int32(IMAX), cnt0))
            nv = (n + 15) >> 4

            def kb(j, c):
                x = cval[pl.ds(j * 16, 16)]
                ks = _keys16(x)
                valid = (j * 16 + lane) < n
                ckey[pl.ds(j * 16, 16)] = jnp.where(valid, ks, jnp.int32(IMIN))
                return c
            lax.fori_loop(0, nv, kb, 0)

            lanebase = lane * 256
            prefix = jnp.int32(0)
            r_cur = jnp.int32(KSEL)
            cnt_gt = jnp.int32(0)
            for lvl in range(4):
                shift = 24 - 8 * lvl

                def zb(j, c):
                    hist[pl.ds(j * 16, 16)] = jnp.zeros((16,), jnp.int32)
                    return c
                lax.fori_loop(0, 256, zb, 0)

                pfx = prefix

                def hb(j, c, _lvl=lvl, _shift=shift, _pfx=pfx):
                    ks = ckey[pl.ds(j * 16, 16)]
                    u = ks ^ jnp.int32(IMIN)
                    b = lax.shift_right_logical(u, _shift) & 255
                    m = (j * 16 + lane) < n
                    if _lvl > 0:
                        m = m & (lax.shift_right_logical(u, _shift + 8) == _pfx)
                    plsc.addupdate_scatter(
                        hist, [lanebase + b], jnp.ones((16,), jnp.int32),
                        mask=m)
                    return c
                lax.fori_loop(0, nv, hb, 0)

                sfx[pl.ds(256, 16)] = jnp.zeros((16,), jnp.int32)

                def tb(t, carry):
                    cchunk = 15 - t

                    def lb(ln, a):
                        return a + hist[pl.ds(ln * 256 + cchunk * 16, 16)]
                    acc = lax.fori_loop(0, 16, lb, jnp.zeros((16,), jnp.int32))
                    cs = _cumsum16(lax.rev(acc, (0,))) + carry
                    sfx[pl.ds(cchunk * 16, 16)] = lax.rev(cs, (0,))
                    return cs[15]
                lax.fori_loop(0, 16, tb, jnp.int32(0))

                def fb(ci, bstar):
                    v = sfx[pl.ds(ci * 16, 16)]
                    idxv = ci * 16 + lane
                    sel = jnp.where(v >= r_cur, idxv, jnp.int32(-1))
                    return jnp.maximum(bstar, _vmax16(sel))
                bstar = lax.fori_loop(0, 16, fb, jnp.int32(-1))
                bstar = jnp.maximum(bstar, 0)
                gt_above = sfx[pl.ds(bstar + 1, 16)][0]
                cnt_gt = cnt_gt + gt_above
                r_cur = r_cur - gt_above
                prefix = (prefix << 8) | bstar

            t_ks = prefix ^ jnp.int32(IMIN)
            eq_need = jnp.int32(KSEL) - cnt_gt

            def sb(j, carry):
                cursor, eq_carry = carry
                ks = ckey[pl.ds(j * 16, 16)]
                valid = (j * 16 + lane) < n
                gt = (ks > t_ks) & valid
                eq = (ks == t_ks) & valid
                ecs = _cumsum16(eq.astype(jnp.int32))
                sel = gt | (eq & ((ecs + eq_carry) <= eq_need))
                seli = sel.astype(jnp.int32)
                dst = cursor + _cumsum16(seli) - seli
                tpos = (dst & 15) * 16 + lax.shift_right_logical(dst, 4)
                sk = ks ^ jnp.int32(0x7FFFFFFF)
                plsc.store_scatter(ask, [tpos], sk, mask=sel)
                plsc.store_scatter(aid, [tpos], cidx[pl.ds(j * 16, 16)],
                                   mask=sel)
                return (cursor + _popcnt(sel), eq_carry + _popcnt(eq))
            lax.fori_loop(0, nv, sb, (jnp.int32(0), jnp.int32(0)))

            padidx = jnp.minimum(175 + lane * 16, 255)
            padm = lane < 6
            plsc.store_scatter(ask, [padidx],
                               jnp.full((16,), -1, jnp.int32), mask=padm)
            plsc.store_scatter(aid, [padidx],
                               jnp.zeros((16,), jnp.int32), mask=padm)

            for half in range(2):
                def gbj(j, c, _h=half):
                    idv = aid[pl.ds(_h * 128 + j * 16, 16)]
                    gid[_h, pl.ds(j * 16, 16)] = idv + row_base
                    return c
                lax.fori_loop(0, 8, gbj, 0)
            cp0 = pltpu.make_async_copy(
                gx_hbm.at[gid.at[0]], gxb.at[pl.ds(0, 128)], semg)
            cp0.start()
            cp1 = pltpu.make_async_copy(
                gx_hbm.at[gid.at[1]], gxb.at[pl.ds(128, 128)], semg)
            cp1.start()
            cp0.wait()
            cp1.wait()

            curtok = curb[pl.ds(rl, 16)][0]

            def pb(j, c):
                idv = aid[pl.ds(j * 16, 16)]
                g = gxb[pl.ds(j * 16, 16)]
                sc = jnp.where(idv == curtok, jnp.float32(EPS_C),
                               jnp.float32(1.0))
                aprop[pl.ds(j * 16, 16)] = (-(g * sc)) / jnp.float32(TEMP_C)
                return c
            lax.fori_loop(0, NSEL // 16, pb, 0)

            bufs = [(ask, aid, aprop), (bsk, bid, bprop)]
            for p in range(8):
                src3 = bufs[p % 2]
                dst3 = bufs[(p + 1) % 2]
                sh = 4 * p
                last = p == 7

                def zb2(t, c):
                    hist[pl.ds(t * 16, 16)] = jnp.zeros((16,), jnp.int32)
                    return c
                lax.fori_loop(0, 16, zb2, 0)

                def hb2(j, c, _sh=sh, _src=src3[0]):
                    sk = _src[pl.ds(j * 16, 16)]
                    d = lax.shift_right_logical(sk, _sh) & 15
                    plsc.addupdate_scatter(hist, [d * 16 + lane],
                                           jnp.ones((16,), jnp.int32))
                    return c
                lax.fori_loop(0, 16, hb2, 0)

                def pb2(t, carry):
                    v = hist[pl.ds(t * 16, 16)]
                    inc = _cumsum16(v)
                    hist[pl.ds(t * 16, 16)] = inc - v + carry
                    return carry + inc[15]
                lax.fori_loop(0, 16, pb2, jnp.int32(0))

                def mb(j, c, _sh=sh, _s=src3, _d=dst3, _last=last):
                    sk = _s[0][pl.ds(j * 16, 16)]
                    dg = lax.shift_right_logical(sk, _sh) & 15
                    addr = dg * 16 + lane
                    pp = plsc.load_gather(hist, [addr])
                    plsc.store_scatter(hist, [addr], pp + 1)
                    if _last:
                        tpos = pp
                    else:
                        tpos = (pp & 15) * 16 + lax.shift_right_logical(pp, 4)
                    plsc.store_scatter(_d[0], [tpos], sk)
                    plsc.store_scatter(_d[1], [tpos],
                                       _s[1][pl.ds(j * 16, 16)])
                    plsc.store_scatter(_d[2], [tpos],
                                       _s[2][pl.ds(j * 16, 16)])
                    return c
                lax.fori_loop(0, 16, mb, 0)

            pltpu.sync_copy(aprop.at[pl.ds(0, NSEL)], prop_hbm.at[row])
            pltpu.sync_copy(aid.at[pl.ds(0, NSEL)], id_hbm.at[row])
            return _carry

        lax.fori_loop(0, RPW, row_body, 0)

    return k(logits_f, gx_f, cur_f)


def kernel(gx, logits, cur_token_ids):
    logits_f = logits.reshape(-1)
    gx_f = gx.reshape(-1)
    cur_f = cur_token_ids.reshape(-1).astype(jnp.int32)
    prop, ids = _sc_topk_call(logits_f, gx_f, cur_f)
    proposal = prop[:, :KSEL].reshape(B_, S_, KSEL)
    topk_ids = ids[:, :KSEL].reshape(B_, S_, KSEL)
    return (proposal, topk_ids)
